# baseline (speedup 1.0000x reference)
import functools

import jax
import jax.numpy as jnp
from jax.experimental import pallas as pl
from jax.experimental.pallas import tpu as pltpu
from jax.experimental.pallas import tpu_sc as plsc

H = 16
D = 64
T = 128
NT = 16
S = T * NT
HD = H * D
SCALE = D ** -0.5
QB = 256


def _mm_kernel(a_ref, b_ref, o_ref):
    o_ref[...] = jnp.dot(a_ref[...], b_ref[...],
                         preferred_element_type=jnp.float32).astype(o_ref.dtype)


def _mm(a, b, bn, out_dtype):
    m, k = a.shape
    _, n = b.shape
    return pl.pallas_call(
        _mm_kernel,
        grid=(n // bn,),
        in_specs=[pl.BlockSpec((m, k), lambda t: (0, 0)),
                  pl.BlockSpec((k, bn), lambda t: (0, t))],
        out_specs=pl.BlockSpec((m, bn), lambda t: (0, t)),
        out_shape=jax.ShapeDtypeStruct((m, n), out_dtype),
    )(a, b)


def _select_kernel(bm_ref, am_ref):
    bm = bm_ref[...]
    r = jax.lax.broadcasted_iota(jnp.int32, (H * NT, NT), 0)
    i_row = jax.lax.rem(r, NT)
    j = jax.lax.broadcasted_iota(jnp.int32, (H * NT, NT), 1)
    active = ((j < i_row) & (bm != 0)) | (j == i_row)
    add = jnp.where(active, 0.0, -1e30).astype(jnp.float32)
    g = (jax.lax.broadcasted_iota(jnp.int32, (NT, S), 0)
         == jax.lax.broadcasted_iota(jnp.int32, (NT, S), 1) // T)
    am_ref[...] = jnp.dot(add, g.astype(jnp.float32),
                          preferred_element_type=jnp.float32)


def _select(bm2):
    return pl.pallas_call(
        _select_kernel,
        out_shape=jax.ShapeDtypeStruct((H * NT, S), jnp.float32),
    )(bm2)


_NW = 32
_RPW = (H * NT) // _NW


def _sc_select(bm2):
    mesh = plsc.VectorSubcoreMesh(core_axis_name="c", subcore_axis_name="s",
                                  num_cores=2, num_subcores=16)

    @functools.partial(
        pl.kernel, mesh=mesh,
        out_type=jax.ShapeDtypeStruct((H * NT, S), jnp.float32),
        scratch_types=[pltpu.VMEM((_RPW, NT), jnp.int32),
                       pltpu.VMEM((NT,), jnp.float32),
                       pltpu.VMEM((_RPW, S), jnp.float32)],
    )
    def sel(bm_hbm, am_hbm, bm_v, add_v, out_v):
        wid = jax.lax.axis_index("s") * 2 + jax.lax.axis_index("c")
        base = wid * _RPW
        pltpu.sync_copy(bm_hbm.at[pl.ds(base, _RPW)], bm_v)
        jvec = jax.lax.iota(jnp.int32, NT)
        for r in range(_RPW):
            i = jax.lax.rem(base + r, NT)
            bmrow = bm_v[r]
            active = ((jvec < i) & (bmrow != 0)) | (jvec == i)
            addv = jnp.where(active, 0.0, jnp.float32(-1e30))
            for j in range(NT):
                splat = jnp.full((NT,), addv[j], jnp.float32)
                for q in range(T // NT):
                    out_v[r, pl.ds(j * T + q * NT, NT)] = splat
        pltpu.sync_copy(out_v, am_hbm.at[pl.ds(base, _RPW)])

    return sel(bm2)


def _attn_body(i0, ext):
    def body(q_ref, k_ref, v_ref, am_ref, o_ref):
        i = pl.program_id(1) + i0
        q = q_ref[0]
        k = k_ref[0]
        s = jax.lax.dot_general(q, k, (((1,), (1,)), ((), ())),
                                preferred_element_type=jnp.float32)
        am = am_ref[:, 0, :]
        amx = jnp.concatenate(
            [jnp.broadcast_to(am[t:t + 1], (T, ext)) for t in range(QB // T)],
            0)
        grow = i * QB + jax.lax.broadcasted_iota(jnp.int32, (QB, ext), 0)
        gcol = jax.lax.broadcasted_iota(jnp.int32, (QB, ext), 1)
        s = jnp.where(gcol <= grow, s + amx, -1e30)
        m = jnp.max(s, axis=1, keepdims=True)
        p = jnp.exp(s - m)
        l = jnp.sum(p, axis=1, keepdims=True)
        o = jnp.dot(p.astype(jnp.bfloat16), v_ref[0],
                    preferred_element_type=jnp.float32) / l
        o_ref[0] = o.astype(o_ref.dtype)
    return body


def _attend_part(am, qkv, i0, nqc, ext):
    nq = NT * T // QB
    return pl.pallas_call(
        _attn_body(i0, ext),
        grid=(H, nqc),
        in_specs=[pl.BlockSpec((1, QB, D), lambda h, i: (h, i0 + i, 0)),
                  pl.BlockSpec((1, ext, D), lambda h, i: (H + h, 0, 0)),
                  pl.BlockSpec((1, ext, D), lambda h, i: (2 * H + h, 0, 0)),
                  pl.BlockSpec((QB // T, 1, ext),
                               lambda h, i: (h * nq + i0 + i, 0, 0))],
        out_specs=pl.BlockSpec((1, QB, D), lambda h, i: (h, i, 0)),
        out_shape=jax.ShapeDtypeStruct((H, nqc * QB, D), jnp.bfloat16),
    )(qkv, qkv, qkv, am)


def _attend(am, qkv):
    parts = [_attend_part(am, qkv, i0, 2, (i0 + 2) * QB) for i0 in (0, 2, 4, 6)]
    return jnp.concatenate(parts, axis=1)


def _oproj_kernel(a_ref, b_ref, o_ref):
    @pl.when(pl.program_id(0) == 0)
    def _():
        o_ref[...] = jnp.zeros_like(o_ref)

    o_ref[...] += jnp.dot(a_ref[0], b_ref[0],
                          preferred_element_type=jnp.float32)


def _oproj(attn, wo3):
    e = wo3.shape[2]
    return pl.pallas_call(
        _oproj_kernel,
        grid=(H,),
        in_specs=[pl.BlockSpec((1, S, D), lambda h: (h, 0, 0)),
                  pl.BlockSpec((1, D, e), lambda h: (h, 0, 0))],
        out_specs=pl.BlockSpec((S, e), lambda h: (0, 0)),
        out_shape=jax.ShapeDtypeStruct((S, e), jnp.float32),
    )(attn, wo3)


def kernel(x, block_mask, Wq, Wk, Wv, Wo):
    batch, _, e = x.shape
    xb = x.reshape(S, e).astype(jnp.bfloat16)
    w = jnp.concatenate([Wq * SCALE, Wk, Wv], axis=1).astype(jnp.bfloat16)
    qkv = _mm(xb, w, 512, jnp.bfloat16)
    qkvt = qkv.reshape(S, 3 * H, D).transpose(1, 0, 2)
    bm2 = block_mask.reshape(H * NT, NT).astype(jnp.int32)
    am = _sc_select(bm2)
    attn = _attend(am.reshape(H * NT, 1, S), qkvt)
    out = _oproj(attn, Wo.reshape(H, D, -1).astype(jnp.bfloat16))
    return out.reshape(batch, S, -1)


# speedup vs baseline: 1.0477x; 1.0477x over previous
import functools

import jax
import jax.numpy as jnp
from jax.experimental import pallas as pl
from jax.experimental.pallas import tpu as pltpu
from jax.experimental.pallas import tpu_sc as plsc

H = 16
D = 64
T = 128
NT = 16
S = T * NT
HD = H * D
SCALE = D ** -0.5
QB = 512


def _mm_kernel(a_ref, b_ref, o_ref):
    o_ref[...] = jnp.dot(a_ref[...], b_ref[...],
                         preferred_element_type=jnp.float32).astype(o_ref.dtype)


def _mm(a, b, bn, out_dtype):
    m, k = a.shape
    _, n = b.shape
    return pl.pallas_call(
        _mm_kernel,
        grid=(n // bn,),
        in_specs=[pl.BlockSpec((m, k), lambda t: (0, 0)),
                  pl.BlockSpec((k, bn), lambda t: (0, t))],
        out_specs=pl.BlockSpec((m, bn), lambda t: (0, t)),
        out_shape=jax.ShapeDtypeStruct((m, n), out_dtype),
    )(a, b)


_NW = 32
_RPW = (H * NT) // _NW


def _sc_select(bm2):
    mesh = plsc.VectorSubcoreMesh(core_axis_name="c", subcore_axis_name="s",
                                  num_cores=2, num_subcores=16)

    @functools.partial(
        pl.kernel, mesh=mesh,
        out_type=jax.ShapeDtypeStruct((H * NT, S), jnp.float32),
        scratch_types=[pltpu.VMEM((_RPW, NT), jnp.int32),
                       pltpu.VMEM((_RPW, S), jnp.float32)],
    )
    def sel(bm_hbm, am_hbm, bm_v, out_v):
        wid = jax.lax.axis_index("s") * 2 + jax.lax.axis_index("c")
        base = wid * _RPW
        pltpu.sync_copy(bm_hbm.at[pl.ds(base, _RPW)], bm_v)
        jvec = jax.lax.iota(jnp.int32, NT)
        for r in range(_RPW):
            i = jax.lax.rem(base + r, NT)
            bmrow = bm_v[r]
            active = ((jvec < i) & (bmrow != 0)) | (jvec == i)
            addv = jnp.where(active, 0.0, jnp.float32(-1e30))
            for j in range(NT):
                splat = jnp.full((NT,), addv[j], jnp.float32)
                for q in range(T // NT):
                    out_v[r, pl.ds(j * T + q * NT, NT)] = splat
        pltpu.sync_copy(out_v, am_hbm.at[pl.ds(base, _RPW)])

    return sel(bm2)


def _attn_body(i0, ext):
    def body(q_ref, k_ref, v_ref, am_ref, o_ref):
        q = q_ref[0]
        am = am_ref[:, 0, :]
        amx = jnp.concatenate(
            [jnp.broadcast_to(am[t:t + 1], (T, ext)) for t in range(QB // T)],
            0)
        r_ = jax.lax.broadcasted_iota(jnp.int32, (QB, QB), 0)
        c_ = jax.lax.broadcasted_iota(jnp.int32, (QB, QB), 1)
        band = jnp.where((r_ // T == c_ // T) & (c_ > r_), -1e30, 0.0)
        half = ext // 2 if ext > QB else ext
        bo = ext - QB - half
        ss = []
        for lo, hi in ((0, half), (half, ext)):
            if lo == hi:
                continue
            sh = jax.lax.dot_general(q, k_ref[0, lo:hi, :],
                                     (((1,), (1,)), ((), ())),
                                     preferred_element_type=jnp.float32)
            sh = sh + amx[:, lo:hi]
            if lo == 0 and ext == QB:
                sh = sh + band
            elif lo > 0:
                if bo > 0:
                    sh = jnp.concatenate([sh[:, :bo], sh[:, bo:] + band], 1)
                else:
                    sh = sh + band
            ss.append(sh)
        m = ss[0].max(axis=1, keepdims=True)
        for sh in ss[1:]:
            m = jnp.maximum(m, sh.max(axis=1, keepdims=True))
        ps = [jnp.exp(sh - m) for sh in ss]
        l = ps[0].sum(axis=1, keepdims=True)
        for ph in ps[1:]:
            l = l + ph.sum(axis=1, keepdims=True)
        o = jnp.zeros((QB, D), jnp.float32)
        for ph, (lo, hi) in zip(ps, ((0, half), (half, ext))):
            o = o + jax.lax.dot_general(
                ph.astype(jnp.bfloat16), v_ref[0, lo:hi, :],
                (((1,), (0,)), ((), ())), preferred_element_type=jnp.float32)
        o_ref[0] = (o / l).astype(o_ref.dtype)
    return body


def _attend_part(am, qkv, i0, nqc, ext):
    nq = NT * T // QB
    return pl.pallas_call(
        _attn_body(i0, ext),
        grid=(H, nqc),
        in_specs=[pl.BlockSpec((1, QB, D), lambda h, i: (h, i0 + i, 0)),
                  pl.BlockSpec((1, ext, D), lambda h, i: (H + h, 0, 0)),
                  pl.BlockSpec((1, ext, D), lambda h, i: (2 * H + h, 0, 0)),
                  pl.BlockSpec((QB // T, 1, ext),
                               lambda h, i: (h * nq + i0 + i, 0, 0))],
        out_specs=pl.BlockSpec((1, QB, D), lambda h, i: (h, i, 0)),
        out_shape=jax.ShapeDtypeStruct((H, nqc * QB, D), jnp.bfloat16),
        compiler_params=pltpu.CompilerParams(
            dimension_semantics=("parallel", "arbitrary")),
    )(qkv, qkv, qkv, am)


def _attend(am, qkv):
    return [_attend_part(am, qkv, i0, 1, (i0 + 1) * QB) for i0 in (0, 1, 2, 3)]


def _oproj_kernel(a0_ref, a1_ref, a2_ref, a3_ref, b_ref, o_ref):
    @pl.when(pl.program_id(0) == 0)
    def _():
        o_ref[...] = jnp.zeros_like(o_ref)

    for p, a_ref in enumerate((a0_ref, a1_ref, a2_ref, a3_ref)):
        rows = a_ref.shape[1]
        o_ref[p * rows:(p + 1) * rows, :] += jnp.dot(
            a_ref[0], b_ref[0], preferred_element_type=jnp.float32)


def _oproj(parts, wo3):
    e = wo3.shape[2]
    rows = parts[0].shape[1]
    part_spec = pl.BlockSpec((1, rows, D), lambda h: (h, 0, 0))
    return pl.pallas_call(
        _oproj_kernel,
        grid=(H,),
        in_specs=[part_spec, part_spec, part_spec, part_spec,
                  pl.BlockSpec((1, D, e), lambda h: (h, 0, 0))],
        out_specs=pl.BlockSpec((S, e), lambda h: (0, 0)),
        out_shape=jax.ShapeDtypeStruct((S, e), jnp.float32),
    )(*parts, wo3)


def kernel(x, block_mask, Wq, Wk, Wv, Wo):
    batch, _, e = x.shape
    xb = x.reshape(S, e).astype(jnp.bfloat16)
    w = jnp.concatenate([Wq * SCALE, Wk, Wv], axis=1).astype(jnp.bfloat16)
    qkv = _mm(xb, w, 512, jnp.bfloat16)
    qkvt = qkv.reshape(S, 3 * H, D).transpose(1, 0, 2)
    bm2 = block_mask.reshape(H * NT, NT).astype(jnp.int32)
    am = _sc_select(bm2)
    parts = _attend(am.reshape(H * NT, 1, S), qkvt)
    out = _oproj(parts, Wo.reshape(H, D, -1).astype(jnp.bfloat16))
    return out.reshape(batch, S, -1)
